# baseline (speedup 1.0000x reference)
import jax
import jax.numpy as jnp
from jax import lax
from jax.experimental import pallas as pl
from jax.experimental.pallas import tpu as pltpu
from jax.experimental.pallas import tpu_sc as plsc

_DIM = 1024
_POS = 77
_NC = 2
_NS = 16
_NW = _NC * _NS
_C = 8
_NBUF = 4
_L = 16
_ROWS = 1024 * 77
_RPW = _ROWS // _NW
_NCH = _RPW // _C


def _body(ids_hbm, tok_hbm, pos_hbm, out_hbm,
          idx_v, pos_v, b0, b1, b2, b3, g0, g1, g2, g3, w0, w1, w2, w3):
    bufs = (b0, b1, b2, b3)
    gsems = (g0, g1, g2, g3)
    wsems = (w0, w1, w2, w3)
    wid = lax.axis_index("s") * _NC + lax.axis_index("c")
    base = wid * _RPW
    pltpu.sync_copy(ids_hbm.at[pl.ds(base, _RPW)], idx_v)
    pltpu.sync_copy(pos_hbm, pos_v)

    def g_desc(c, b):
        off = pl.multiple_of(c * _C, 8)
        return pltpu.make_async_copy(
            tok_hbm.at[idx_v.at[pl.ds(off, _C)]], bufs[b], gsems[b])

    def w_desc(c, b):
        off = pl.multiple_of(base + c * _C, 8)
        return pltpu.make_async_copy(
            bufs[b], out_hbm.at[pl.ds(off, _C)], wsems[b])

    def add_pos(c, b):
        buf = bufs[b]
        prow = lax.rem(c * _C, _POS)
        ps = [lax.rem(prow + j, _POS) for j in range(_C)]

        def o_body(o, carry):
            sl = pl.ds(o * _L, _L)
            for j in range(_C):
                buf[j, sl] = buf[j, sl] + pos_v[ps[j], sl]
            return carry

        lax.fori_loop(0, _DIM // _L, o_body, 0, unroll=2)

    def step(c, b, wait_prev_write, start_next_gather):
        bg = (b + 3) % _NBUF
        if start_next_gather:
            if wait_prev_write:
                w_desc(c - 1, bg).wait()
            g_desc(c + 3, bg).start()
        g_desc(c, b).wait()
        add_pos(c, b)
        w_desc(c, b).start()

    for b in range(_NBUF - 1):
        g_desc(b, b).start()

    for b in range(_NBUF):
        step(b, b, wait_prev_write=(b >= 1), start_next_gather=True)

    def block(blk, carry):
        g = blk * _NBUF
        for b in range(_NBUF):
            step(g + b, b, wait_prev_write=True, start_next_gather=True)
        return carry

    lax.fori_loop(1, _NCH // _NBUF - 1, block, 0)

    g_last = _NCH - _NBUF
    for b in range(_NBUF):
        c = g_last + b
        step(c, b, wait_prev_write=(b == 0), start_next_gather=(b == 0))
    for b in range(_NBUF):
        w_desc(g_last + b, b).wait()


_embed = pl.kernel(
    _body,
    out_type=jax.ShapeDtypeStruct((_ROWS, _DIM), jnp.float32),
    mesh=plsc.VectorSubcoreMesh(core_axis_name="c", subcore_axis_name="s"),
    scratch_types=[
        pltpu.VMEM((_RPW,), jnp.int32),
        pltpu.VMEM((_POS, _DIM), jnp.float32),
    ] + [pltpu.VMEM((_C, _DIM), jnp.float32) for _ in range(_NBUF)]
      + [pltpu.SemaphoreType.DMA for _ in range(2 * _NBUF)],
)


def kernel(input_ids, token_table, position_table):
    b, s = input_ids.shape
    ids = input_ids.reshape(b * s).astype(jnp.int32)
    out = _embed(ids, token_table, position_table)
    return out.reshape(b, s, _DIM)


# speedup vs baseline: 1.7752x; 1.7752x over previous
import jax
import jax.numpy as jnp
from jax import lax
from jax.experimental import pallas as pl
from jax.experimental.pallas import tpu as pltpu
from jax.experimental.pallas import tpu_sc as plsc

_B = 1024
_S = 77
_SP = 80
_D = 1024
_L = 16
_NC = 2
_NS = 16
_NW = _NC * _NS
_BPW = _B // _NW
_CH = 8
_NSLOT = _SP // _CH // 2
_PHR = _NSLOT * _CH


def _body(ids_hbm, tok_hbm, pos_hbm, out_hbm,
          idx_v, pos_v,
          g0, g1, g2, g3, g4, w0, w1, w2, w3, w4,
          gs0, gs1, gs2, gs3, gs4, ws0, ws1, ws2, ws3, ws4):
    gbufs = (g0, g1, g2, g3, g4)
    wbufs = (w0, w1, w2, w3, w4)
    gsems = (gs0, gs1, gs2, gs3, gs4)
    wsems = (ws0, ws1, ws2, ws3, ws4)
    wid = lax.axis_index("s") * _NC + lax.axis_index("c")
    b0 = wid * _BPW
    pltpu.sync_copy(ids_hbm.at[pl.ds(b0 * _SP, _BPW * _SP)], idx_v)

    def g_desc(idx_off, j):
        off = pl.multiple_of(idx_off, 8)
        return pltpu.make_async_copy(
            tok_hbm.at[idx_v.at[pl.ds(off, _CH)]], gbufs[j], gsems[j])

    def w_desc(b, ph, j):
        s0 = ph * _PHR + j * _CH
        rows = 5 if (ph == 1 and j == _NSLOT - 1) else _CH
        src = wbufs[j] if rows == _CH else wbufs[j].at[pl.ds(0, rows)]
        return pltpu.make_async_copy(
            src, out_hbm.at[b, pl.ds(s0, rows)], wsems[j])

    def add_chunk(j):
        gbuf, wbuf = gbufs[j], wbufs[j]

        def o_body(o, carry):
            sl = pl.ds(o * _L, _L)
            for jj in range(_CH):
                wbuf[jj, sl] = gbuf[jj, sl] + pos_v[j * _CH + jj, sl]
            return carry

        lax.fori_loop(0, _D // _L, o_body, 0)

    def slot(ph, prev_ph, b_l, j, wait_write, start_gather,
             gather_next_phase):
        b = b0 + b_l
        if wait_write:
            w_desc(b0, prev_ph, j).wait()
        g_desc(b_l * _SP + ph * _PHR + j * _CH, j).wait()
        add_chunk(j)
        w_desc(b, ph, j).start()
        if start_gather:
            nph = ph + 1 if gather_next_phase else ph
            nb = 0 if gather_next_phase else b_l + 1
            g_desc(nb * _SP + nph * _PHR + j * _CH, j).start()

    pltpu.sync_copy(pos_hbm.at[pl.ds(0, _PHR)], pos_v)
    for j in range(_NSLOT):
        g_desc(0 * _SP + 0 * _PHR + j * _CH, j).start()

    for ph in range(2):
        if ph == 1:
            pltpu.sync_copy(pos_hbm.at[pl.ds(_PHR, _PHR)], pos_v)
        for j in range(_NSLOT):
            slot(ph, 0, 0, j, wait_write=(ph == 1), start_gather=True,
                 gather_next_phase=False)

        def bat_body(b_l, carry):
            for j in range(_NSLOT):
                slot(ph, ph, b_l, j, wait_write=True, start_gather=True,
                     gather_next_phase=False)
            return carry

        lax.fori_loop(1, _BPW - 1, bat_body, 0)

        for j in range(_NSLOT):
            slot(ph, ph, _BPW - 1, j, wait_write=True,
                 start_gather=(ph == 0), gather_next_phase=True)

    for j in range(_NSLOT):
        w_desc(b0 + _BPW - 1, 1, j).wait()


_embed = pl.kernel(
    _body,
    out_type=jax.ShapeDtypeStruct((_B, _S, _D), jnp.float32),
    mesh=plsc.VectorSubcoreMesh(core_axis_name="c", subcore_axis_name="s"),
    scratch_types=[
        pltpu.VMEM((_BPW * _SP,), jnp.int32),
        pltpu.VMEM((_PHR, _D), jnp.float32),
    ] + [pltpu.VMEM((_CH, _D), jnp.float32) for _ in range(2 * _NSLOT)]
      + [pltpu.SemaphoreType.DMA for _ in range(2 * _NSLOT)],
)


def kernel(input_ids, token_table, position_table):
    ids = jnp.pad(input_ids.astype(jnp.int32), ((0, 0), (0, _SP - _S)))
    pos = jnp.pad(position_table, ((0, _SP - _S), (0, 0)))
    return _embed(ids.reshape(_B * _SP), token_table, pos)


# speedup vs baseline: 3.7393x; 2.1063x over previous
import jax
import jax.numpy as jnp
from jax import lax
from jax.experimental import pallas as pl
from jax.experimental.pallas import tpu as pltpu
from jax.experimental.pallas import tpu_sc as plsc

_B = 1024
_S = 77
_D = 1024
_L = 16
_NC = 2
_NS = 16
_NW = _NC * _NS
_BB = 4
_CH = 8
_PHR = 40
_IPW = _BB * _S * _CH


def _body(ids_hbm, tok_hbm, pos_hbm, out_hbm,
          idx_v, pos_v,
          g0, g1, g2, g3, w0, w1, w2, w3,
          gs0, gs1, gs2, gs3, ws0, ws1, ws2, ws3):
    gbufs = (g0, g1, g2, g3)
    wbufs = (w0, w1, w2, w3)
    gsems = (gs0, gs1, gs2, gs3)
    wsems = (ws0, ws1, ws2, ws3)
    wid = lax.axis_index("s") * _NC + lax.axis_index("c")
    b0 = wid * (_BB * _CH)
    pltpu.sync_copy(ids_hbm.at[pl.ds(wid * _IPW, _IPW)], idx_v)

    def g_desc(s, bb):
        off = pl.multiple_of(bb * (_S * _CH) + s * _CH, 8)
        return pltpu.make_async_copy(
            tok_hbm.at[idx_v.at[pl.ds(off, _CH)]], gbufs[bb], gsems[bb])

    def w_desc(s, bb):
        return pltpu.make_async_copy(
            wbufs[bb], out_hbm.at[s, pl.ds(b0 + bb * _CH, _CH)], wsems[bb])

    def add_unit(srow, bb):
        gbuf, wbuf = gbufs[bb], wbufs[bb]

        def o_body(o, carry):
            sl = pl.ds(o * _L, _L)
            pv = pos_v[srow, sl]
            for jj in range(_CH):
                wbuf[jj, sl] = gbuf[jj, sl] + pv
            return carry

        lax.fori_loop(0, _D // _L, o_body, 0)

    def unit(ph, s, bb, wait_write, start_gather):
        if wait_write:
            w_desc(s, bb).wait()
        g_desc(s, bb).wait()
        add_unit(s - ph * _PHR, bb)
        w_desc(s, bb).start()
        if start_gather:
            g_desc(s + 1, bb).start()

    pltpu.sync_copy(pos_hbm.at[pl.ds(0, _PHR)], pos_v)
    for bb in range(_BB):
        g_desc(0, bb).start()
    for bb in range(_BB):
        unit(0, 0, bb, wait_write=False, start_gather=True)

    def s_body(ph):
        def body(s, carry):
            for bb in range(_BB):
                unit(ph, s, bb, wait_write=True, start_gather=True)
            return carry
        return body

    lax.fori_loop(1, _PHR, s_body(0), 0)

    pltpu.sync_copy(pos_hbm.at[pl.ds(_PHR, _PHR)], pos_v)
    lax.fori_loop(_PHR, _S - 1, s_body(1), 0)

    for bb in range(_BB):
        unit(1, _S - 1, bb, wait_write=True, start_gather=False)
    for bb in range(_BB):
        w_desc(_S - 1, bb).wait()


_embed = pl.kernel(
    _body,
    out_type=jax.ShapeDtypeStruct((_S, _B, _D), jnp.float32),
    mesh=plsc.VectorSubcoreMesh(core_axis_name="c", subcore_axis_name="s"),
    scratch_types=[
        pltpu.VMEM((_IPW,), jnp.int32),
        pltpu.VMEM((_PHR, _D), jnp.float32),
    ] + [pltpu.VMEM((_CH, _D), jnp.float32) for _ in range(8)]
      + [pltpu.SemaphoreType.DMA for _ in range(8)],
)


def kernel(input_ids, token_table, position_table):
    ids = (input_ids.astype(jnp.int32)
           .reshape(_B // _CH, _CH, _S).transpose(0, 2, 1).reshape(-1))
    pos = jnp.pad(position_table, ((0, 2 * _PHR - _S), (0, 0)))
    out = _embed(ids, token_table, pos)
    return out.transpose(1, 0, 2)


# speedup vs baseline: 5.2036x; 1.3916x over previous
import jax
import jax.numpy as jnp
from jax import lax
from jax.experimental import pallas as pl
from jax.experimental.pallas import tpu as pltpu
from jax.experimental.pallas import tpu_sc as plsc

_B = 1024
_S = 77
_D = 1024
_L = 16
_NC = 2
_NS = 16
_NW = _NC * _NS
_BB = 4
_CH = 8
_PHR = 40
_IPW = _BB * _S * _CH


def _body(ids_hbm, tok_hbm, pos_hbm, out_hbm,
          idx_v, pos_v,
          g0, g1, g2, g3, w0, w1, w2, w3,
          gs0, gs1, gs2, gs3, ws0, ws1, ws2, ws3):
    gbufs = (g0, g1, g2, g3)
    wbufs = (w0, w1, w2, w3)
    gsems = (gs0, gs1, gs2, gs3)
    wsems = (ws0, ws1, ws2, ws3)
    wid = lax.axis_index("s") * _NC + lax.axis_index("c")
    b0 = wid * (_BB * _CH)
    pltpu.sync_copy(ids_hbm.at[pl.ds(wid * _IPW, _IPW)], idx_v)

    def g_desc(s, bb):
        off = pl.multiple_of(bb * (_S * _CH) + s * _CH, 8)
        return pltpu.make_async_copy(
            tok_hbm.at[idx_v.at[pl.ds(off, _CH)]], gbufs[bb], gsems[bb])

    def w_desc(s, bb):
        return pltpu.make_async_copy(
            wbufs[bb], out_hbm.at[s, pl.ds(b0 + bb * _CH, _CH)], wsems[bb])

    def add_unit(srow, bb):
        gbuf, wbuf = gbufs[bb], wbufs[bb]

        @plsc.parallel_loop(0, _D, step=_L, unroll=4)
        def _o_body(o):
            sl = pl.ds(pl.multiple_of(o, _L), _L)
            pv = pos_v[srow, sl]
            for jj in range(_CH):
                wbuf[jj, sl] = gbuf[jj, sl] + pv

    def unit(ph, s, bb, wait_write, start_gather):
        if wait_write:
            w_desc(s, bb).wait()
        g_desc(s, bb).wait()
        add_unit(s - ph * _PHR, bb)
        w_desc(s, bb).start()
        if start_gather:
            g_desc(s + 1, bb).start()

    pltpu.sync_copy(pos_hbm.at[pl.ds(0, _PHR)], pos_v)
    for bb in range(_BB):
        g_desc(0, bb).start()
    for bb in range(_BB):
        unit(0, 0, bb, wait_write=False, start_gather=True)

    def s_body(ph):
        def body(s, carry):
            for bb in range(_BB):
                unit(ph, s, bb, wait_write=True, start_gather=True)
            return carry
        return body

    lax.fori_loop(1, _PHR, s_body(0), 0)

    pltpu.sync_copy(pos_hbm.at[pl.ds(_PHR, _PHR)], pos_v)
    lax.fori_loop(_PHR, _S - 1, s_body(1), 0)

    for bb in range(_BB):
        unit(1, _S - 1, bb, wait_write=True, start_gather=False)
    for bb in range(_BB):
        w_desc(_S - 1, bb).wait()


_embed = pl.kernel(
    _body,
    out_type=jax.ShapeDtypeStruct((_S, _B, _D), jnp.float32),
    mesh=plsc.VectorSubcoreMesh(core_axis_name="c", subcore_axis_name="s"),
    scratch_types=[
        pltpu.VMEM((_IPW,), jnp.int32),
        pltpu.VMEM((_PHR, _D), jnp.float32),
    ] + [pltpu.VMEM((_CH, _D), jnp.float32) for _ in range(8)]
      + [pltpu.SemaphoreType.DMA for _ in range(8)],
)


def kernel(input_ids, token_table, position_table):
    ids = (input_ids.astype(jnp.int32)
           .reshape(_B // _CH, _CH, _S).transpose(0, 2, 1).reshape(-1))
    pos = jnp.pad(position_table, ((0, 2 * _PHR - _S), (0, 0)))
    out = _embed(ids, token_table, pos)
    return out.transpose(1, 0, 2)
